# baseline (speedup 1.0000x reference)
import functools

import jax
import jax.numpy as jnp
from jax import lax
from jax.experimental import pallas as pl
from jax.experimental.pallas import tpu as pltpu
from jax.experimental.pallas import tpu_sc as plsc

N = 10000
E = 320000
D = 128

NC = 2
NS = 16
NW = NC * NS
CHUNK = 128
NCHUNKS = E // CHUNK
BASE = NCHUNKS // NW
EXTRA = NCHUNKS - BASE * NW
NPAD = 10240
SEG = NPAD // NS
LANES = 16


def _indeg_body(ei_hbm, out0_hbm, out1_hbm, idx_v, idx_x, ones_v, zeros_v, acc_sh):
    c = lax.axis_index("c")
    s = lax.axis_index("s")
    wid = s * NC + c

    for j in range(CHUNK // LANES):
        ones_v[pl.ds(j * LANES, LANES)] = jnp.full((LANES,), 1.0, jnp.float32)
    for j in range(SEG // LANES):
        zeros_v[pl.ds(j * LANES, LANES)] = jnp.zeros((LANES,), jnp.float32)

    pltpu.sync_copy(ei_hbm.at[0, pl.ds(wid * BASE, BASE)], idx_v)

    @pl.when(wid < EXTRA)
    def _():
        pltpu.sync_copy(ei_hbm.at[0, NW * BASE + wid], idx_x)

    pltpu.sync_copy(zeros_v, acc_sh.at[pl.ds(s * SEG, SEG)])
    plsc.subcore_barrier()

    def chunk(j, carry):
        pltpu.sync_copy(ones_v, acc_sh.at[idx_v.at[j]], add=True)
        return carry

    lax.fori_loop(0, BASE, chunk, 0, unroll=False)

    @pl.when(wid < EXTRA)
    def _():
        pltpu.sync_copy(ones_v, acc_sh.at[idx_x], add=True)

    plsc.subcore_barrier()

    @pl.when(c == 0)
    def _():
        pltpu.sync_copy(acc_sh.at[pl.ds(s * SEG, SEG)], out0_hbm.at[pl.ds(s * SEG, SEG)])

    @pl.when(c == 1)
    def _():
        pltpu.sync_copy(acc_sh.at[pl.ds(s * SEG, SEG)], out1_hbm.at[pl.ds(s * SEG, SEG)])


_indeg_kernel = functools.partial(
    pl.kernel,
    out_type=(
        jax.ShapeDtypeStruct((NPAD,), jnp.float32),
        jax.ShapeDtypeStruct((NPAD,), jnp.float32),
    ),
    mesh=plsc.VectorSubcoreMesh(core_axis_name="c", subcore_axis_name="s"),
    compiler_params=pltpu.CompilerParams(use_tc_tiling_on_sc=False),
    scratch_types=[
        pltpu.VMEM((BASE, CHUNK), jnp.int32),
        pltpu.VMEM((CHUNK,), jnp.int32),
        pltpu.VMEM((CHUNK,), jnp.float32),
        pltpu.VMEM((SEG,), jnp.float32),
        pltpu.VMEM_SHARED((NPAD,), jnp.float32),
    ],
)(_indeg_body)


BLK = 1024
CROWS = BLK // CHUNK


def _proj_body(x_ref, c0_ref, c1_ref, Wv_ref, bv_ref, Wo_ref, bo_ref, o_ref):
    v = jnp.dot(x_ref[...], Wv_ref[...], preferred_element_type=jnp.float32) + bv_ref[...]
    cnt = c0_ref[...] + c1_ref[...]
    q = lax.broadcasted_iota(jnp.int32, (BLK, CHUNK), 0)
    l = lax.broadcasted_iota(jnp.int32, (BLK, CHUNK), 1)
    rowsel = (lax.broadcasted_iota(jnp.int32, (BLK, CROWS), 0) // CHUNK
              == lax.broadcasted_iota(jnp.int32, (BLK, CROWS), 1)).astype(jnp.float32)
    a = jnp.dot(rowsel, cnt, preferred_element_type=jnp.float32)
    msum = jnp.sum(jnp.where(l == q % CHUNK, a, 0.0), axis=1, keepdims=True)
    h = jnp.where(msum > 0.0, v, 0.0)
    o_ref[...] = jnp.dot(h, Wo_ref[...], preferred_element_type=jnp.float32) + bo_ref[...]


def _proj(x, c0, c1, Wv, bv, Wo, bo):
    return pl.pallas_call(
        _proj_body,
        grid=(NPAD // BLK,),
        in_specs=[
            pl.BlockSpec((BLK, D), lambda i: (i, 0)),
            pl.BlockSpec((CROWS, CHUNK), lambda i: (i, 0)),
            pl.BlockSpec((CROWS, CHUNK), lambda i: (i, 0)),
            pl.BlockSpec((D, D), lambda i: (0, 0)),
            pl.BlockSpec((1, D), lambda i: (0, 0)),
            pl.BlockSpec((D, D), lambda i: (0, 0)),
            pl.BlockSpec((1, D), lambda i: (0, 0)),
        ],
        out_specs=pl.BlockSpec((BLK, D), lambda i: (i, 0)),
        out_shape=jax.ShapeDtypeStruct((N, D), jnp.float32),
    )(x, c0, c1, Wv, bv, Wo, bo)


def kernel(node_feats, edge_feats, edge_index, Wq, bq, Wk, bk, Wv, bv, Wb, bb, Wo, bo):
    dest = edge_index[1].astype(jnp.int32).reshape(1, NCHUNKS, CHUNK)
    cnt0, cnt1 = _indeg_kernel(dest)
    return _proj(node_feats, cnt0.reshape(NPAD // CHUNK, CHUNK),
                 cnt1.reshape(NPAD // CHUNK, CHUNK),
                 Wv, bv.reshape(1, D), Wo, bo.reshape(1, D))


# speedup vs baseline: 1.2406x; 1.2406x over previous
import functools

import jax
import jax.numpy as jnp
from jax import lax
from jax.experimental import pallas as pl
from jax.experimental.pallas import tpu as pltpu
from jax.experimental.pallas import tpu_sc as plsc

N = 10000
E = 320000
D = 128

NC = 2
NS = 16
NW = NC * NS
CHUNK = 128
NCHUNKS = E // CHUNK
BASE = NCHUNKS // NW
EXTRA = NCHUNKS - BASE * NW
NPAD = 10240
SEG = NPAD // NS
LANES = 16


def _indeg_body(ei_hbm, out0_hbm, out1_hbm, idx_v, idx_x, ones_v, zeros_v, acc_sh):
    c = lax.axis_index("c")
    s = lax.axis_index("s")
    wid = s * NC + c

    for j in range(CHUNK // LANES):
        ones_v[pl.ds(j * LANES, LANES)] = jnp.full((LANES,), 1.0, jnp.float32)
    for j in range(SEG // LANES):
        zeros_v[pl.ds(j * LANES, LANES)] = jnp.zeros((LANES,), jnp.float32)

    pltpu.sync_copy(ei_hbm.at[1, pl.ds(wid * BASE, BASE)], idx_v)

    @pl.when(wid < EXTRA)
    def _():
        pltpu.sync_copy(ei_hbm.at[1, NW * BASE + wid], idx_x)

    pltpu.sync_copy(zeros_v, acc_sh.at[pl.ds(s * SEG, SEG)])
    plsc.subcore_barrier()

    def chunk(j, carry):
        pltpu.sync_copy(ones_v, acc_sh.at[idx_v.at[j]], add=True)
        return carry

    lax.fori_loop(0, BASE, chunk, 0, unroll=False)

    @pl.when(wid < EXTRA)
    def _():
        pltpu.sync_copy(ones_v, acc_sh.at[idx_x], add=True)

    plsc.subcore_barrier()

    @pl.when(c == 0)
    def _():
        pltpu.sync_copy(acc_sh.at[pl.ds(s * SEG, SEG)], out0_hbm.at[pl.ds(s * SEG, SEG)])

    @pl.when(c == 1)
    def _():
        pltpu.sync_copy(acc_sh.at[pl.ds(s * SEG, SEG)], out1_hbm.at[pl.ds(s * SEG, SEG)])


_indeg_kernel = functools.partial(
    pl.kernel,
    out_type=(
        jax.ShapeDtypeStruct((NPAD,), jnp.float32),
        jax.ShapeDtypeStruct((NPAD,), jnp.float32),
    ),
    mesh=plsc.VectorSubcoreMesh(core_axis_name="c", subcore_axis_name="s"),
    compiler_params=pltpu.CompilerParams(use_tc_tiling_on_sc=False),
    scratch_types=[
        pltpu.VMEM((BASE, CHUNK), jnp.int32),
        pltpu.VMEM((CHUNK,), jnp.int32),
        pltpu.VMEM((CHUNK,), jnp.float32),
        pltpu.VMEM((SEG,), jnp.float32),
        pltpu.VMEM_SHARED((NPAD,), jnp.float32),
    ],
)(_indeg_body)


BLK = 1024
CROWS = BLK // CHUNK


def _proj_body(x_ref, c0_ref, c1_ref, Wv_ref, bv_ref, Wo_ref, bo_ref, o_ref):
    v = jnp.dot(x_ref[...], Wv_ref[...], preferred_element_type=jnp.float32) + bv_ref[...]
    cnt = c0_ref[...] + c1_ref[...]
    q = lax.broadcasted_iota(jnp.int32, (BLK, CHUNK), 0)
    l = lax.broadcasted_iota(jnp.int32, (BLK, CHUNK), 1)
    rowsel = (lax.broadcasted_iota(jnp.int32, (BLK, CROWS), 0) // CHUNK
              == lax.broadcasted_iota(jnp.int32, (BLK, CROWS), 1)).astype(jnp.float32)
    a = jnp.dot(rowsel, cnt, preferred_element_type=jnp.float32)
    msum = jnp.sum(jnp.where(l == q % CHUNK, a, 0.0), axis=1, keepdims=True)
    h = jnp.where(msum > 0.0, v, 0.0)
    o_ref[...] = jnp.dot(h, Wo_ref[...], preferred_element_type=jnp.float32) + bo_ref[...]


def _proj(x, c0, c1, Wv, bv, Wo, bo):
    return pl.pallas_call(
        _proj_body,
        grid=(NPAD // BLK,),
        in_specs=[
            pl.BlockSpec((BLK, D), lambda i: (i, 0)),
            pl.BlockSpec((CROWS, CHUNK), lambda i: (i, 0)),
            pl.BlockSpec((CROWS, CHUNK), lambda i: (i, 0)),
            pl.BlockSpec((D, D), lambda i: (0, 0)),
            pl.BlockSpec((1, D), lambda i: (0, 0)),
            pl.BlockSpec((D, D), lambda i: (0, 0)),
            pl.BlockSpec((1, D), lambda i: (0, 0)),
        ],
        out_specs=pl.BlockSpec((BLK, D), lambda i: (i, 0)),
        out_shape=jax.ShapeDtypeStruct((N, D), jnp.float32),
    )(x, c0, c1, Wv, bv, Wo, bo)


def kernel(node_feats, edge_feats, edge_index, Wq, bq, Wk, bk, Wv, bv, Wb, bb, Wo, bo):
    ei = edge_index.astype(jnp.int32).reshape(2, NCHUNKS, CHUNK)
    cnt0, cnt1 = _indeg_kernel(ei)
    return _proj(node_feats, cnt0.reshape(NPAD // CHUNK, CHUNK),
                 cnt1.reshape(NPAD // CHUNK, CHUNK),
                 Wv, bv.reshape(1, D), Wo, bo.reshape(1, D))


# speedup vs baseline: 1.3931x; 1.1229x over previous
import functools

import jax
import jax.numpy as jnp
from jax import lax
from jax.experimental import pallas as pl
from jax.experimental.pallas import tpu as pltpu
from jax.experimental.pallas import tpu_sc as plsc

N = 10000
E = 320000
D = 128

NC = 2
NS = 16
NW = NC * NS
CHUNK = 128
NCHUNKS = E // CHUNK
BASE = NCHUNKS // NW
EXTRA = NCHUNKS - BASE * NW
NPAD = 10240
SEG = NPAD // NS
LANES = 16


WCHUNK = BASE * CHUNK


def _indeg_body(ei_hbm, out0_hbm, out1_hbm, idx_v, idx_x, ones_v, zeros_v, acc_sh):
    c = lax.axis_index("c")
    s = lax.axis_index("s")
    wid = s * NC + c

    for j in range(CHUNK // LANES):
        ones_v[pl.ds(j * LANES, LANES)] = jnp.full((LANES,), 1.0, jnp.float32)
    for j in range(SEG // LANES):
        zeros_v[pl.ds(j * LANES, LANES)] = jnp.zeros((LANES,), jnp.float32)

    pltpu.sync_copy(ei_hbm.at[:, pl.ds(wid * WCHUNK, WCHUNK)], idx_v)

    @pl.when(wid < EXTRA)
    def _():
        pltpu.sync_copy(ei_hbm.at[:, pl.ds(NW * WCHUNK + wid * CHUNK, CHUNK)], idx_x)

    pltpu.sync_copy(zeros_v, acc_sh.at[pl.ds(s * SEG, SEG)])
    plsc.subcore_barrier()

    def chunk(j, carry):
        pltpu.sync_copy(ones_v, acc_sh.at[idx_v.at[1, pl.ds(j * CHUNK, CHUNK)]], add=True)
        return carry

    lax.fori_loop(0, BASE, chunk, 0, unroll=False)

    @pl.when(wid < EXTRA)
    def _():
        pltpu.sync_copy(ones_v, acc_sh.at[idx_x.at[1]], add=True)

    plsc.subcore_barrier()

    @pl.when(c == 0)
    def _():
        pltpu.sync_copy(acc_sh.at[pl.ds(s * SEG, SEG)], out0_hbm.at[pl.ds(s * SEG, SEG)])

    @pl.when(c == 1)
    def _():
        pltpu.sync_copy(acc_sh.at[pl.ds(s * SEG, SEG)], out1_hbm.at[pl.ds(s * SEG, SEG)])


_indeg_kernel = functools.partial(
    pl.kernel,
    out_type=(
        jax.ShapeDtypeStruct((NPAD,), jnp.float32),
        jax.ShapeDtypeStruct((NPAD,), jnp.float32),
    ),
    mesh=plsc.VectorSubcoreMesh(core_axis_name="c", subcore_axis_name="s"),
    scratch_types=[
        pltpu.VMEM((2, WCHUNK), jnp.int32),
        pltpu.VMEM((2, CHUNK), jnp.int32),
        pltpu.VMEM((CHUNK,), jnp.float32),
        pltpu.VMEM((SEG,), jnp.float32),
        pltpu.VMEM_SHARED((NPAD,), jnp.float32),
    ],
)(_indeg_body)


BLK = 2048
CROWS = BLK // CHUNK


def _proj_body(x_ref, c0_ref, c1_ref, Wv_ref, bv_ref, Wo_ref, bo_ref, o_ref):
    v = jnp.dot(x_ref[...], Wv_ref[...], preferred_element_type=jnp.float32) + bv_ref[...]
    cnt = c0_ref[...] + c1_ref[...]
    q = lax.broadcasted_iota(jnp.int32, (BLK, CHUNK), 0)
    l = lax.broadcasted_iota(jnp.int32, (BLK, CHUNK), 1)
    rowsel = (lax.broadcasted_iota(jnp.int32, (BLK, CROWS), 0) // CHUNK
              == lax.broadcasted_iota(jnp.int32, (BLK, CROWS), 1)).astype(jnp.float32)
    a = jnp.dot(rowsel, cnt, preferred_element_type=jnp.float32)
    msum = jnp.sum(jnp.where(l == q % CHUNK, a, 0.0), axis=1, keepdims=True)
    h = jnp.where(msum > 0.0, v, 0.0)
    o_ref[...] = jnp.dot(h, Wo_ref[...], preferred_element_type=jnp.float32) + bo_ref[...]


def _proj(x, c0, c1, Wv, bv, Wo, bo):
    return pl.pallas_call(
        _proj_body,
        grid=(NPAD // BLK,),
        in_specs=[
            pl.BlockSpec((BLK, D), lambda i: (i, 0)),
            pl.BlockSpec((CROWS, CHUNK), lambda i: (i, 0)),
            pl.BlockSpec((CROWS, CHUNK), lambda i: (i, 0)),
            pl.BlockSpec((D, D), lambda i: (0, 0)),
            pl.BlockSpec((1, D), lambda i: (0, 0)),
            pl.BlockSpec((D, D), lambda i: (0, 0)),
            pl.BlockSpec((1, D), lambda i: (0, 0)),
        ],
        out_specs=pl.BlockSpec((BLK, D), lambda i: (i, 0)),
        out_shape=jax.ShapeDtypeStruct((N, D), jnp.float32),
    )(x, c0, c1, Wv, bv, Wo, bo)


def kernel(node_feats, edge_feats, edge_index, Wq, bq, Wk, bk, Wv, bv, Wb, bb, Wo, bo):
    ei = edge_index.astype(jnp.int32)
    cnt0, cnt1 = _indeg_kernel(ei)
    return _proj(node_feats, cnt0.reshape(NPAD // CHUNK, CHUNK),
                 cnt1.reshape(NPAD // CHUNK, CHUNK),
                 Wv, bv.reshape(1, D), Wo, bo.reshape(1, D))
